# baseline (speedup 1.0000x reference)
import jax
import jax.numpy as jnp
from jax.experimental import pallas as pl
from jax.experimental.pallas import tpu as pltpu

_P = 152


def _decode_body(x_ref, tab_ref, mxy_ref, o_ref):
    x = x_ref[...]
    sg = tab_ref[0][None]
    av = tab_ref[1][None]
    bv = tab_ref[2][None]
    dv = tab_ref[3][None]
    ev = tab_ref[4][None]
    e = jnp.exp(x * sg)
    s = 1.0 / (1.0 + e)
    mx = mxy_ref[:, :, 0:1]
    my = mxy_ref[:, :, 1:2]
    res = av * s + bv * e + dv * mx + ev * my
    for aa in range(3):
        piece = res[:, :, 85 * aa:85 * (aa + 1)]
        o_ref[:, aa] = jnp.transpose(piece, (1, 0, 2))


def kernel(raw, anchors, img_size):
    nB, nCHA, nG, _ = raw.shape
    nA = anchors.shape[0]
    nCH = nCHA // nA
    L = nG * nG
    f32 = jnp.float32
    stride = (img_size // nG).astype(f32) if hasattr(img_size, "astype") \
        else f32(img_size // nG)
    c2 = jnp.arange(nA * nCH, dtype=jnp.int32)
    c = c2 % nCH
    aidx = c2 // nCH
    anch = anchors.astype(f32)[aidx]
    is_wh = jnp.logical_or(c == 2, c == 3)
    sg = jnp.where(is_wh, f32(1), f32(-1))
    av = jnp.where(is_wh, f32(0), jnp.where(c < 2, stride, f32(1)))
    bv = jnp.where(c == 2, anch[:, 0], jnp.where(c == 3, anch[:, 1], f32(0)))
    dv = jnp.where(c == 0, stride, f32(0))
    ev = jnp.where(c == 1, stride, f32(0))
    tab = jnp.broadcast_to(
        jnp.stack([sg, av, bv, dv, ev])[:, None, :], (5, nB, nA * nCH))
    p = jnp.arange(L, dtype=jnp.int32)
    mxy = jnp.broadcast_to(
        jnp.stack([(p % nG).astype(f32), (p // nG).astype(f32)],
                  axis=-1)[:, None, :], (L, nB, 2))
    x3 = jnp.transpose(raw, (2, 3, 0, 1)).reshape(L, nB, nA * nCH)
    out = pl.pallas_call(
        _decode_body,
        grid=(L // _P,),
        in_specs=[
            pl.BlockSpec((_P, nB, nA * nCH), lambda j: (j, 0, 0)),
            pl.BlockSpec((5, nB, nA * nCH), lambda j: (0, 0, 0)),
            pl.BlockSpec((_P, nB, 2), lambda j: (j, 0, 0)),
        ],
        out_specs=pl.BlockSpec((nB, nA, _P, nCH), lambda j: (0, 0, j, 0)),
        out_shape=jax.ShapeDtypeStruct((nB, nA, L, nCH), jnp.float32),
        compiler_params=pltpu.CompilerParams(
            dimension_semantics=("arbitrary",)),
    )(x3, tab, mxy)
    return out.reshape(nB, nA * L, nCH)


# speedup vs baseline: 1.0407x; 1.0407x over previous
import jax
import jax.numpy as jnp
from jax.experimental import pallas as pl
from jax.experimental.pallas import tpu as pltpu

_P = 304


def _decode_body(x_ref, tab_ref, mxy_ref, o_ref):
    x = x_ref[...]
    sg = tab_ref[0][None]
    av = tab_ref[1][None]
    bv = tab_ref[2][None]
    dv = tab_ref[3][None]
    ev = tab_ref[4][None]
    e = jnp.exp(x * sg)
    s = 1.0 / (1.0 + e)
    mx = mxy_ref[:, :, 0:1]
    my = mxy_ref[:, :, 1:2]
    res = av * s + bv * e + dv * mx + ev * my
    for aa in range(3):
        piece = res[:, :, 85 * aa:85 * (aa + 1)]
        o_ref[:, aa] = jnp.transpose(piece, (1, 0, 2))


def kernel(raw, anchors, img_size):
    nB, nCHA, nG, _ = raw.shape
    nA = anchors.shape[0]
    nCH = nCHA // nA
    L = nG * nG
    f32 = jnp.float32
    stride = (img_size // nG).astype(f32) if hasattr(img_size, "astype") \
        else f32(img_size // nG)
    c2 = jnp.arange(nA * nCH, dtype=jnp.int32)
    c = c2 % nCH
    aidx = c2 // nCH
    anch = anchors.astype(f32)[aidx]
    is_wh = jnp.logical_or(c == 2, c == 3)
    sg = jnp.where(is_wh, f32(1), f32(-1))
    av = jnp.where(is_wh, f32(0), jnp.where(c < 2, stride, f32(1)))
    bv = jnp.where(c == 2, anch[:, 0], jnp.where(c == 3, anch[:, 1], f32(0)))
    dv = jnp.where(c == 0, stride, f32(0))
    ev = jnp.where(c == 1, stride, f32(0))
    tab = jnp.broadcast_to(
        jnp.stack([sg, av, bv, dv, ev])[:, None, :], (5, nB, nA * nCH))
    p = jnp.arange(L, dtype=jnp.int32)
    mxy = jnp.broadcast_to(
        jnp.stack([(p % nG).astype(f32), (p // nG).astype(f32)],
                  axis=-1)[:, None, :], (L, nB, 2))
    x3 = jnp.transpose(raw, (2, 3, 0, 1)).reshape(L, nB, nA * nCH)
    out = pl.pallas_call(
        _decode_body,
        grid=(L // _P,),
        in_specs=[
            pl.BlockSpec((_P, nB, nA * nCH), lambda j: (j, 0, 0)),
            pl.BlockSpec((5, nB, nA * nCH), lambda j: (0, 0, 0)),
            pl.BlockSpec((_P, nB, 2), lambda j: (j, 0, 0)),
        ],
        out_specs=pl.BlockSpec((nB, nA, _P, nCH), lambda j: (0, 0, j, 0)),
        out_shape=jax.ShapeDtypeStruct((nB, nA, L, nCH), jnp.float32),
        compiler_params=pltpu.CompilerParams(
            dimension_semantics=("arbitrary",)),
    )(x3, tab, mxy)
    return out.reshape(nB, nA * L, nCH)
